# baseline (speedup 1.0000x reference)
import jax
import jax.numpy as jnp
from jax.experimental import pallas as pl
from jax.experimental.pallas import tpu as pltpu

_F32 = jnp.float32


def _lrelu(v):
    return jnp.maximum(v, 0.01 * v)


def _dotg(lhs, rhs, dims):
    return jax.lax.dot_general(lhs, rhs, (dims, ((), ())),
                               preferred_element_type=_F32)


def _gcn_body(at_ref, x_ref, k1t_ref, k2_ref, b1_ref, b2_ref, o_ref):
    at = at_ref[...]
    x = x_ref[0]
    p1t = _dotg(x, at, ((0,), (0,)))
    h1t = _lrelu(_dotg(k1t_ref[...], p1t, ((1,), (0,))) + b1_ref[...])
    p2t = _dotg(h1t, at, ((1,), (0,)))
    h2 = _lrelu(_dotg(p2t, k2_ref[...], ((0,), (0,))) + b2_ref[...])
    o_ref[0] = h2


def kernel(X, A, W1, b1, W2, b2):
    B, N, T, F_in = X.shape
    F_sp = W1.shape[1]
    assert F_in == F_sp, "flattened-column layout assumes F_in == F_sp"
    S = T * F_in

    Xr = X.reshape(B, N, S)
    At = A.T
    eye = jnp.eye(T, dtype=X.dtype)
    K1t = jnp.kron(eye, W1.T)
    K2 = jnp.kron(eye, W2)
    b1c = jnp.tile(b1, T)[:, None]
    b2r = jnp.tile(b2, T)[None, :]

    out = pl.pallas_call(
        _gcn_body,
        grid=(B,),
        in_specs=[
            pl.BlockSpec((N, N), lambda b: (0, 0)),
            pl.BlockSpec((1, N, S), lambda b: (b, 0, 0)),
            pl.BlockSpec((S, S), lambda b: (0, 0)),
            pl.BlockSpec((S, S), lambda b: (0, 0)),
            pl.BlockSpec((S, 1), lambda b: (0, 0)),
            pl.BlockSpec((1, S), lambda b: (0, 0)),
        ],
        out_specs=pl.BlockSpec((1, N, S), lambda b: (b, 0, 0)),
        out_shape=jax.ShapeDtypeStruct((B, N, S), _F32),
        compiler_params=pltpu.CompilerParams(
            dimension_semantics=("arbitrary",),
        ),
    )(At, Xr, K1t, K2, b1c, b2r)

    return out.reshape(B, N, T, F_sp)


# speedup vs baseline: 1.1169x; 1.1169x over previous
import jax
import jax.numpy as jnp
from jax.experimental import pallas as pl
from jax.experimental.pallas import tpu as pltpu

_BPS = 8
_KW = 128
_F32 = jnp.float32
_BF = jnp.bfloat16


def _lrelu(v):
    return jnp.maximum(v, 0.01 * v)


def _chain(a, xw, k1, k2, b1, b2, S, o_ref):
    W = xw.shape[1]
    p1 = jnp.dot(a, xw, preferred_element_type=_F32)
    hs = []
    for c in range(W // _KW):
        h = jnp.dot(p1[:, c * _KW:(c + 1) * _KW], k1, preferred_element_type=_F32)
        hs.append(_lrelu(h + b1).astype(_BF))
    h1 = jnp.concatenate(hs, axis=1)
    p2 = jnp.dot(a, h1, preferred_element_type=_F32)
    for i in range(_BPS):
        sl = p2[:, i * S:(i + 1) * S]
        h = jnp.dot(sl, k2, preferred_element_type=_F32)
        o_ref[i] = _lrelu(h + b2)


def _gcn_body(a_ref, x_ref, k1_ref, k2_ref, b1_ref, b2_ref, o_ref,
              xw0_ref, xw1_ref):
    j = pl.program_id(0)
    a = a_ref[...]
    S = x_ref.shape[2]

    @pl.when(j == 0)
    def _():
        xw0_ref[...] = jnp.concatenate(
            [x_ref[i] for i in range(_BPS)], axis=1).astype(_BF)

    @pl.when(j == 1)
    def _():
        xw1_ref[...] = jnp.concatenate(
            [x_ref[i] for i in range(_BPS)], axis=1).astype(_BF)

    @pl.when(j == 1)
    def _():
        _chain(a, xw0_ref[...], k1_ref[...], k2_ref[...],
               b1_ref[...], b2_ref[...], S, o_ref)

    @pl.when(j == 2)
    def _():
        _chain(a, xw1_ref[...], k1_ref[...], k2_ref[...],
               b1_ref[...], b2_ref[...], S, o_ref)


def kernel(X, A, W1, b1, W2, b2):
    B, N, T, F_in = X.shape
    F_sp = W1.shape[1]
    assert F_in == F_sp, "flattened-column layout assumes F_in == F_sp"
    S = T * F_in

    Xr = X.reshape(B, N, S)
    nblk = _KW // F_in
    K1 = jnp.kron(jnp.eye(nblk, dtype=X.dtype), W1)
    K2 = jnp.kron(jnp.eye(T, dtype=X.dtype), W2)
    b1t = jnp.tile(b1, nblk)[None, :]
    b2t = jnp.tile(b2, T)[None, :]

    n_tiles = B // _BPS
    out = pl.pallas_call(
        _gcn_body,
        grid=(n_tiles + 1,),
        in_specs=[
            pl.BlockSpec((N, N), lambda j: (0, 0)),
            pl.BlockSpec((_BPS, N, S), lambda j: (jnp.minimum(j, 1), 0, 0)),
            pl.BlockSpec((_KW, _KW), lambda j: (0, 0)),
            pl.BlockSpec((S, S), lambda j: (0, 0)),
            pl.BlockSpec((1, _KW), lambda j: (0, 0)),
            pl.BlockSpec((1, S), lambda j: (0, 0)),
        ],
        out_specs=pl.BlockSpec(
            (_BPS, N, S), lambda j: (jnp.maximum(j - 1, 0), 0, 0)),
        out_shape=jax.ShapeDtypeStruct((B, N, S), _F32),
        scratch_shapes=[
            pltpu.VMEM((N, _BPS * S), _BF),
            pltpu.VMEM((N, _BPS * S), _BF),
        ],
        compiler_params=pltpu.CompilerParams(
            dimension_semantics=("arbitrary",),
        ),
    )(A.astype(_BF), Xr, K1, K2, b1t, b2t)

    return out.reshape(B, N, T, F_sp)
